# baseline (speedup 1.0000x reference)
import functools

import jax
import jax.numpy as jnp
from jax import lax
from jax.experimental import pallas as pl
from jax.experimental.pallas import tpu as pltpu
from jax.experimental.pallas import tpu_sc as plsc

DIM = 128
BATCH = 16384

NC = 2
NS = 16
NW = NC * NS
B_PER_W = BATCH // NW


NUM_HOURS = 24
SIZES = (64, 384, 64)
STARTS = (0, 64, 448)
NCHUNK = len(SIZES)
CH = max(SIZES)
IDX_SPLIT = 64


def _make_lookup():
    mesh = plsc.VectorSubcoreMesh(core_axis_name="c", subcore_axis_name="s")

    @functools.partial(
        pl.kernel,
        mesh=mesh,
        out_type=jax.ShapeDtypeStruct((BATCH, DIM), jnp.float32),
        scratch_types=[
            pltpu.VMEM((B_PER_W,), jnp.int32),
            pltpu.VMEM((CH, DIM), jnp.float32),
            pltpu.VMEM((CH, DIM), jnp.float32),
            pltpu.VMEM_SHARED((NUM_HOURS, DIM), jnp.float32),
            pltpu.SemaphoreType.DMA,
            pltpu.SemaphoreType.DMA,
            pltpu.SemaphoreType.DMA,
            pltpu.SemaphoreType.DMA,
            pltpu.SemaphoreType.DMA,
            pltpu.SemaphoreType.DMA,
        ],
    )
    def k(table_hbm, idx_hbm, out_hbm, idx_v, rows0, rows1, table_sh,
          g0, g1, o0, o1, i0, i1):
        sid = lax.axis_index("s")
        wid = sid * NC + lax.axis_index("c")
        base = wid * B_PER_W
        idx_copies = [
            pltpu.async_copy(
                idx_hbm.at[pl.ds(base, IDX_SPLIT)],
                idx_v.at[pl.ds(0, IDX_SPLIT)],
                i0,
            ),
            pltpu.async_copy(
                idx_hbm.at[pl.ds(base + IDX_SPLIT, B_PER_W - IDX_SPLIT)],
                idx_v.at[pl.ds(IDX_SPLIT, B_PER_W - IDX_SPLIT)],
                i1,
            ),
        ]
        @pl.when(sid < NUM_HOURS // 2)
        def _():
            pltpu.sync_copy(
                table_hbm.at[pl.ds(sid * 2, 2)], table_sh.at[pl.ds(sid * 2, 2)]
            )

        plsc.subcore_barrier()

        bufs = (rows0, rows1)
        gsems = (g0, g1)
        osems = (o0, o1)
        gathers = [None] * NCHUNK
        outs = [None] * NCHUNK
        idx_waited = [False, False]
        for c in range(NCHUNK):
            b = c % 2
            if c >= 2:
                outs[c - 2].wait()
            need = 0 if STARTS[c] + SIZES[c] <= IDX_SPLIT else 1
            for w in range(need + 1):
                if not idx_waited[w]:
                    idx_copies[w].wait()
                    idx_waited[w] = True
            gathers[c] = pltpu.async_copy(
                table_sh.at[idx_v.at[pl.ds(STARTS[c], SIZES[c])]],
                bufs[b].at[pl.ds(0, SIZES[c])],
                gsems[b],
            )
            gathers[c].wait()
            outs[c] = pltpu.async_copy(
                bufs[b].at[pl.ds(0, SIZES[c])],
                out_hbm.at[pl.ds(base + STARTS[c], SIZES[c])],
                osems[b],
            )
        outs[NCHUNK - 2].wait()
        outs[NCHUNK - 1].wait()

    return k


_lookup = _make_lookup()


def kernel(hour, hour_table):
    idx = hour.astype(jnp.int32)
    return _lookup(hour_table, idx)


# speedup vs baseline: 1.0129x; 1.0129x over previous
import functools

import jax
import jax.numpy as jnp
from jax import lax
from jax.experimental import pallas as pl
from jax.experimental.pallas import tpu as pltpu
from jax.experimental.pallas import tpu_sc as plsc

DIM = 128
BATCH = 16384

NC = 2
NS = 16
NW = NC * NS
B_PER_W = BATCH // NW


NUM_HOURS = 24
CH = 256
NCHUNK = B_PER_W // CH
SIZES = (CH,) * NCHUNK
STARTS = tuple(c * CH for c in range(NCHUNK))
IDX_SPLIT = CH


def _make_lookup():
    mesh = plsc.VectorSubcoreMesh(core_axis_name="c", subcore_axis_name="s")

    @functools.partial(
        pl.kernel,
        mesh=mesh,
        out_type=jax.ShapeDtypeStruct((BATCH, DIM), jnp.float32),
        scratch_types=[
            pltpu.VMEM((B_PER_W,), jnp.int32),
            pltpu.VMEM((CH, DIM), jnp.float32),
            pltpu.VMEM((CH, DIM), jnp.float32),
            pltpu.VMEM_SHARED((NUM_HOURS, DIM), jnp.float32),
            pltpu.SemaphoreType.DMA,
            pltpu.SemaphoreType.DMA,
            pltpu.SemaphoreType.DMA,
            pltpu.SemaphoreType.DMA,
            pltpu.SemaphoreType.DMA,
            pltpu.SemaphoreType.DMA,
        ],
    )
    def k(table_hbm, idx_hbm, out_hbm, idx_v, rows0, rows1, table_sh,
          g0, g1, o0, o1, i0, i1):
        sid = lax.axis_index("s")
        wid = sid * NC + lax.axis_index("c")
        base = wid * B_PER_W
        idx_copies = [
            pltpu.async_copy(
                idx_hbm.at[pl.ds(base, IDX_SPLIT)],
                idx_v.at[pl.ds(0, IDX_SPLIT)],
                i0,
            ),
            pltpu.async_copy(
                idx_hbm.at[pl.ds(base + IDX_SPLIT, B_PER_W - IDX_SPLIT)],
                idx_v.at[pl.ds(IDX_SPLIT, B_PER_W - IDX_SPLIT)],
                i1,
            ),
        ]
        @pl.when(sid < NUM_HOURS // 2)
        def _():
            pltpu.sync_copy(
                table_hbm.at[pl.ds(sid * 2, 2)], table_sh.at[pl.ds(sid * 2, 2)]
            )

        plsc.subcore_barrier()

        bufs = (rows0, rows1)
        gsems = (g0, g1)
        osems = (o0, o1)
        gathers = [None] * NCHUNK
        outs = [None] * NCHUNK
        idx_waited = [False, False]
        for c in range(NCHUNK):
            b = c % 2
            if c >= 2:
                outs[c - 2].wait()
            need = 0 if STARTS[c] + SIZES[c] <= IDX_SPLIT else 1
            for w in range(need + 1):
                if not idx_waited[w]:
                    idx_copies[w].wait()
                    idx_waited[w] = True
            gathers[c] = pltpu.async_copy(
                table_sh.at[idx_v.at[pl.ds(STARTS[c], SIZES[c])]],
                bufs[b].at[pl.ds(0, SIZES[c])],
                gsems[b],
            )
            gathers[c].wait()
            outs[c] = pltpu.async_copy(
                bufs[b].at[pl.ds(0, SIZES[c])],
                out_hbm.at[pl.ds(base + STARTS[c], SIZES[c])],
                osems[b],
            )
        outs[NCHUNK - 2].wait()
        outs[NCHUNK - 1].wait()

    return k


_lookup = _make_lookup()


def kernel(hour, hour_table):
    idx = hour.astype(jnp.int32)
    return _lookup(hour_table, idx)
